# baseline (speedup 1.0000x reference)
import functools

import jax
import jax.numpy as jnp
from jax import lax
from jax.experimental import pallas as pl
from jax.experimental.pallas import tpu as pltpu
from jax.experimental.pallas import tpu_sc as plsc

_CH = 128
_G = 1
_NB = 5
_VB = 24576


def _tc_build_table(e0t, e1t, w0, w1, bias, v, dout):
    dims = (((0,), (0,)), ((), ()))

    def body(e0_ref, e1_ref, w0_ref, w1_ref, b_ref, t_ref):
        a0 = lax.dot_general(
            e0_ref[...].astype(jnp.bfloat16), w0_ref[...],
            dimension_numbers=dims, preferred_element_type=jnp.float32,
        )
        a1 = lax.dot_general(
            e1_ref[...].astype(jnp.bfloat16), w1_ref[...],
            dimension_numbers=dims, preferred_element_type=jnp.float32,
        )
        t_ref[...] = jnp.maximum(a0 + a1 + b_ref[...], 0.0)

    d = e0t.shape[0]
    return pl.pallas_call(
        body,
        grid=(pl.cdiv(v, _VB),),
        in_specs=[
            pl.BlockSpec((d, _VB), lambda i: (0, i)),
            pl.BlockSpec((d, _VB), lambda i: (0, i)),
            pl.BlockSpec((d, dout), lambda i: (0, 0)),
            pl.BlockSpec((d, dout), lambda i: (0, 0)),
            pl.BlockSpec((1, dout), lambda i: (0, 0)),
        ],
        out_specs=pl.BlockSpec((_VB, dout), lambda i: (i, 0)),
        out_shape=jax.ShapeDtypeStruct((v, dout), jnp.float32),
    )(e0t, e1t, w0, w1, bias)


def _sc_gather(idx3, table, n, d):
    nw, n_ch, _ = idx3.shape
    per_w = n // nw
    n_grp = n_ch // _G
    grp_rows = _G * _CH
    mesh = plsc.VectorSubcoreMesh(core_axis_name="c", subcore_axis_name="s")
    nc = mesh.num_cores

    @functools.partial(
        pl.kernel,
        out_type=jax.ShapeDtypeStruct((n, d), jnp.float32),
        mesh=mesh,
        scratch_types=[
            pltpu.VMEM((n_ch, _CH), jnp.int32),
            pltpu.VMEM((_NB, grp_rows, d), jnp.float32),
        ] + [pltpu.SemaphoreType.DMA] * (2 * _NB),
    )
    def gather_kernel(idx_hbm, t_hbm, g_hbm, idx_all, rows, *sems):
        wid = lax.axis_index("s") * nc + lax.axis_index("c")
        base = wid * per_w
        sgs = sems[:_NB]
        sws = sems[_NB:]
        pltpu.sync_copy(idx_hbm.at[wid], idx_all)

        def fire_group(grp, buf):
            for j in range(_G):
                pltpu.async_copy(
                    t_hbm.at[idx_all.at[grp * _G + j]],
                    rows.at[buf, pl.ds(j * _CH, _CH)],
                    sgs[buf],
                )

        def wait_group_gathers(buf):
            for j in range(_G):
                pltpu.make_async_copy(
                    t_hbm.at[idx_all.at[0]],
                    rows.at[buf, pl.ds(j * _CH, _CH)],
                    sgs[buf],
                ).wait()

        def wait_group_wb(grp, buf):
            pltpu.make_async_copy(
                rows.at[buf], g_hbm.at[pl.ds(base + grp * grp_rows, grp_rows)],
                sws[buf],
            ).wait()

        for p in range(_NB - 1):
            fire_group(p, p)

        @pl.loop(0, n_grp // _NB)
        def _(h):
            for cur in range(_NB):
                grp = h * _NB + cur
                nxt = (cur + _NB - 1) % _NB

                @pl.when(grp + _NB - 1 < n_grp)
                def _():
                    @pl.when(grp >= 1)
                    def _():
                        wait_group_wb(grp - 1, nxt)

                    fire_group(grp + _NB - 1, nxt)

                wait_group_gathers(cur)
                pltpu.async_copy(
                    rows.at[cur],
                    g_hbm.at[pl.ds(base + grp * grp_rows, grp_rows)],
                    sws[cur],
                )

        for t in range(_NB, 0, -1):
            wait_group_wb(n_grp - t, (n_grp - t) % _NB)

    return gather_kernel(idx3, table)


def kernel(pad_ques, emb0, emb1, W, b):
    B, L = pad_ques.shape
    n = B * L
    v, d0 = emb0.shape
    dout = W.shape[0]
    info = plsc.get_sparse_core_info()
    nw = info.num_cores * info.num_subcores
    w0 = W[:, :d0].T.astype(jnp.bfloat16)
    w1 = W[:, d0:].T.astype(jnp.bfloat16)
    table = _tc_build_table(
        emb0.T, emb1.T, w0, w1, b.reshape(1, dout), v, dout
    )
    idx3 = pad_ques.reshape(nw, n // (nw * _CH), _CH)
    out = _sc_gather(idx3, table, n, dout)
    return out.reshape(B, L, dout)
